# Initial kernel scaffold; baseline (speedup 1.0000x reference)
import jax
import jax.numpy as jnp
from jax.experimental import pallas as pl


def kernel(x, edge_index, edge_weights, W1, b1, W2, b2, W3, b3, W4, b4, W5, b5):
    raise NotImplementedError("write your pallas kernel here")


# speedup vs baseline: 13.2252x; 13.2252x over previous
import functools

import jax
import jax.numpy as jnp
from jax import lax
from jax.experimental import pallas as pl
from jax.experimental.pallas import tpu as pltpu
from jax.experimental.pallas import tpu_sc as plsc

N = 100000
NP = 102400
E = 1600000
EP = 1638400
C = 16
K = 1024
NSUB = 16
NCORE = 2

_MESH = plsc.VectorSubcoreMesh(core_axis_name="c", subcore_axis_name="s")
_SC_PARAMS = pltpu.CompilerParams(use_tc_tiling_on_sc=False)


def _zero_spmem_rows(zv, acc, rbase, nrows, zrows):
    zero16 = jnp.zeros((C,), jnp.float32)

    def fill(i, _):
        zv[i, :] = zero16
        return ()

    lax.fori_loop(0, zrows, fill, ())

    def copy(j, _):
        pltpu.sync_copy(zv, acc.at[pl.ds(rbase + j * zrows, zrows)])
        return ()

    lax.fori_loop(0, nrows // zrows, copy, ())


def _zero_spmem_flat(zv, acc, base, count, zlen):
    zero16 = jnp.zeros((16,), jnp.float32)

    def fill(i, _):
        zv[pl.ds(i * 16, 16)] = zero16
        return ()

    lax.fori_loop(0, zlen // 16, fill, ())

    def copy(j, _):
        pltpu.sync_copy(zv, acc.at[pl.ds(base + j * zlen, zlen)])
        return ()

    lax.fori_loop(0, count // zlen, copy, ())


def _make_degree_kernel():
    per_tile = EP // (NCORE * NSUB)
    nwin = per_tile // K
    zlen = 1600

    @functools.partial(
        pl.kernel,
        mesh=_MESH,
        out_type=(
            jax.ShapeDtypeStruct((NP,), jnp.float32),
            jax.ShapeDtypeStruct((NP,), jnp.float32),
        ),
        scratch_types=[
            pltpu.VMEM((K,), jnp.int32),
            pltpu.VMEM((K,), jnp.float32),
            pltpu.VMEM((zlen,), jnp.float32),
            pltpu.VMEM_SHARED((NP,), jnp.float32),
        ],
        compiler_params=_SC_PARAMS,
    )
    def deg_kernel(dst_hbm, ew_hbm, out0, out1, dstv, ewv, zv, acc):
        cid = lax.axis_index("c")
        sid = lax.axis_index("s")
        _zero_spmem_flat(zv, acc, sid * (NP // NSUB), NP // NSUB, zlen)
        plsc.subcore_barrier()

        ebase = cid * (EP // NCORE) + sid * per_tile

        def win(w, _):
            base = ebase + w * K
            pltpu.sync_copy(dst_hbm.at[pl.ds(base, K)], dstv)
            pltpu.sync_copy(ew_hbm.at[pl.ds(base, K)], ewv)
            pltpu.sync_copy(ewv, acc.at[dstv], add=True)
            return ()

        lax.fori_loop(0, nwin, win, ())
        plsc.subcore_barrier()

        rows = NP // NSUB
        @pl.when(cid == 0)
        def _():
            pltpu.sync_copy(acc.at[pl.ds(sid * rows, rows)],
                            out0.at[pl.ds(sid * rows, rows)])

        @pl.when(cid == 1)
        def _():
            pltpu.sync_copy(acc.at[pl.ds(sid * rows, rows)],
                            out1.at[pl.ds(sid * rows, rows)])

    return deg_kernel


def _make_msg_kernel(split: bool):
    per_tile = EP // (NSUB * (NCORE if split else 1))
    nwin = per_tile // K
    zrows = 64
    groups = K // 16

    @functools.partial(
        pl.kernel,
        mesh=_MESH,
        out_type=(
            jax.ShapeDtypeStruct((NP, C), jnp.float32),
            jax.ShapeDtypeStruct((NP, C), jnp.float32),
        ),
        scratch_types=[
            pltpu.VMEM((K,), jnp.int32),
            pltpu.VMEM((K,), jnp.int32),
            pltpu.VMEM((K,), jnp.float32),
            pltpu.VMEM((K, C), jnp.float32),
            pltpu.VMEM((zrows, C), jnp.float32),
            pltpu.VMEM_SHARED((NP, C), jnp.float32),
            pltpu.SemaphoreType.DMA,
        ],
        compiler_params=_SC_PARAMS,
    )
    def msg_kernel(hA, hB, src_hbm, dst_hbm, ew_hbm, out0, out1,
                   srcv, dstv, ewv, rowsv, zv, acc, gsem):
        cid = lax.axis_index("c")
        sid = lax.axis_index("s")
        _zero_spmem_rows(zv, acc, sid * (NP // NSUB), NP // NSUB, zrows)
        plsc.subcore_barrier()

        if split:
            ebase = cid * (EP // NCORE) + sid * per_tile
        else:
            ebase = sid * per_tile

        iota16 = lax.iota(jnp.int32, 16)

        def win(w, _):
            base = ebase + w * K
            pltpu.sync_copy(src_hbm.at[pl.ds(base, K)], srcv)
            pltpu.sync_copy(dst_hbm.at[pl.ds(base, K)], dstv)
            pltpu.sync_copy(ew_hbm.at[pl.ds(base, K)], ewv)

            if split:
                pltpu.async_copy(hA.at[srcv], rowsv, gsem).wait()
            else:
                @pl.when(cid == 0)
                def _():
                    pltpu.async_copy(hA.at[srcv], rowsv, gsem).wait()

                @pl.when(cid == 1)
                def _():
                    pltpu.async_copy(hB.at[srcv], rowsv, gsem).wait()

            def grp(g, _):
                e0 = g * 16
                ew16 = ewv[pl.ds(e0, 16)]
                for j in range(16):
                    row = rowsv[e0 + j, :]
                    rowsv[e0 + j, :] = row * ew16[j]
                return ()

            lax.fori_loop(0, groups, grp, ())
            pltpu.sync_copy(rowsv, acc.at[dstv], add=True)
            return ()

        lax.fori_loop(0, nwin, win, ())
        plsc.subcore_barrier()

        rows = NP // NSUB
        @pl.when(cid == 0)
        def _():
            pltpu.sync_copy(acc.at[pl.ds(sid * rows, rows)],
                            out0.at[pl.ds(sid * rows, rows)])

        @pl.when(cid == 1)
        def _():
            pltpu.sync_copy(acc.at[pl.ds(sid * rows, rows)],
                            out1.at[pl.ds(sid * rows, rows)])

    return msg_kernel


_DEG = _make_degree_kernel()
_MSG_FULL = _make_msg_kernel(split=False)
_MSG_SPLIT = _make_msg_kernel(split=True)


def _chunk(hp):
    F = hp.shape[1]
    nc = -(-F // C)
    out = []
    for k in range(nc):
        blk = hp[:, k * C:(k + 1) * C]
        if blk.shape[1] < C:
            blk = jnp.pad(blk, ((0, 0), (0, C - blk.shape[1])))
        out.append(blk)
    return out


def _msg_pass(hp, src, dst, ew):
    F = hp.shape[1]
    chunks = _chunk(hp)
    accs = []
    i = 0
    while i < len(chunks):
        if i + 1 < len(chunks):
            o0, o1 = _MSG_FULL(chunks[i], chunks[i + 1], src, dst, ew)
            accs.extend([o0, o1])
            i += 2
        else:
            o0, o1 = _MSG_SPLIT(chunks[i], chunks[i], src, dst, ew)
            accs.append(o0 + o1)
            i += 1
    return jnp.concatenate(accs, axis=1)[:, :F]


def kernel(x, edge_index, edge_weights, W1, b1, W2, b2, W3, b3, W4, b4, W5, b5):
    pad_idx = (jnp.arange(EP - E, dtype=jnp.int32) * 97) % N
    src = jnp.concatenate([edge_index[0].astype(jnp.int32), pad_idx])
    dst = jnp.concatenate([edge_index[1].astype(jnp.int32), pad_idx])
    ew = jnp.concatenate(
        [edge_weights, jnp.zeros((EP - E,), edge_weights.dtype)])

    p0, p1 = _DEG(dst, ew)
    deg = p0 + p1 + 1.0
    dinv = lax.rsqrt(deg)

    xp = jnp.pad(x, ((0, NP - N), (0, 0)))
    h = xp * W1[0][None, :]
    hp = dinv[:, None] * h
    biases = (b1, b2, b3, b4, b5)
    next_W = (W2, W3, W4, W5, None)
    for bl, Wl in zip(biases, next_W):
        acc = _msg_pass(hp, src, dst, ew)
        out = dinv[:, None] * (acc + hp) + bl[None, :]
        if Wl is None:
            return jax.nn.relu(out[:N])
        hp = dinv[:, None] * (out @ Wl)


# speedup vs baseline: 16.6576x; 1.2595x over previous
import functools

import jax
import jax.numpy as jnp
from jax import lax
from jax.experimental import pallas as pl
from jax.experimental.pallas import tpu as pltpu
from jax.experimental.pallas import tpu_sc as plsc

N = 100000
NP = 102400
E = 1600000
EP = 1638400
C = 16
K = 640
KD = 3200
NSUB = 16
NCORE = 2

_MESH = plsc.VectorSubcoreMesh(core_axis_name="c", subcore_axis_name="s")
_SC_PARAMS = pltpu.CompilerParams(use_tc_tiling_on_sc=False)


def _zero_spmem_rows(zv, acc, rbase, nrows, zrows):
    zero16 = jnp.zeros((C,), jnp.float32)

    def fill(i, _):
        zv[i, :] = zero16
        return ()

    lax.fori_loop(0, zrows, fill, ())

    def copy(j, _):
        pltpu.sync_copy(zv, acc.at[pl.ds(rbase + j * zrows, zrows)])
        return ()

    lax.fori_loop(0, nrows // zrows, copy, ())


def _zero_spmem_flat(zv, acc, base, count, zlen):
    zero16 = jnp.zeros((16,), jnp.float32)

    def fill(i, _):
        zv[pl.ds(i * 16, 16)] = zero16
        return ()

    lax.fori_loop(0, zlen // 16, fill, ())

    def copy(j, _):
        pltpu.sync_copy(zv, acc.at[pl.ds(base + j * zlen, zlen)])
        return ()

    lax.fori_loop(0, count // zlen, copy, ())


def _make_degree_kernel():
    per_tile = EP // (NCORE * NSUB)
    nwin = per_tile // KD
    zlen = 1600

    @functools.partial(
        pl.kernel,
        mesh=_MESH,
        out_type=(
            jax.ShapeDtypeStruct((NP,), jnp.float32),
            jax.ShapeDtypeStruct((NP,), jnp.float32),
        ),
        scratch_types=[
            pltpu.VMEM((2, KD), jnp.int32),
            pltpu.VMEM((2, KD), jnp.float32),
            pltpu.VMEM((2, KD), jnp.int32),
            pltpu.VMEM((2, KD), jnp.float32),
            pltpu.VMEM((zlen,), jnp.float32),
            pltpu.VMEM_SHARED((NP,), jnp.float32),
            pltpu.SemaphoreType.DMA,
            pltpu.SemaphoreType.DMA,
            pltpu.SemaphoreType.DMA,
            pltpu.SemaphoreType.DMA,
        ],
        compiler_params=_SC_PARAMS,
    )
    def deg_kernel(dst_hbm, ew_hbm, out0, out1, dstv, ewv, dsts, ews, zv,
                   acc, lsem0, lsem1, ssem0, ssem1):
        cid = lax.axis_index("c")
        sid = lax.axis_index("s")
        _zero_spmem_flat(zv, acc, sid * (NP // NSUB), NP // NSUB, zlen)
        plsc.subcore_barrier()

        ebase = cid * (EP // NCORE) + sid * per_tile
        lsems = (lsem0, lsem1)
        ssems = (ssem0, ssem1)

        def l_descs(w, b):
            base = ebase + w * KD
            return (
                pltpu.make_async_copy(dst_hbm.at[pl.ds(base, KD)],
                                      dstv.at[b], lsems[b]),
                pltpu.make_async_copy(ew_hbm.at[pl.ds(base, KD)],
                                      ewv.at[b], lsems[b]),
            )

        def s_desc(b):
            return pltpu.make_async_copy(ews.at[b], acc.at[dsts.at[b]],
                                         ssems[b])

        for d in l_descs(0, 0):
            d.start()
        for d in l_descs(1, 1):
            d.start()

        def pair(i, _):
            for b in (0, 1):
                w = 2 * i + b
                for d in l_descs(w, b):
                    d.wait()

                @pl.when(w >= 2)
                def _():
                    s_desc(b).wait()

                def cp(j, _):
                    dsts[b, pl.ds(j * 16, 16)] = dstv[b, pl.ds(j * 16, 16)]
                    ews[b, pl.ds(j * 16, 16)] = ewv[b, pl.ds(j * 16, 16)]
                    return ()

                lax.fori_loop(0, KD // 16, cp, ())
                pltpu.async_copy(ews.at[b], acc.at[dsts.at[b]], ssems[b],
                                 add=True)

                @pl.when(w + 2 < nwin)
                def _():
                    for d in l_descs(w + 2, b):
                        d.start()
            return ()

        lax.fori_loop(0, nwin // 2, pair, ())
        s_desc(0).wait()
        s_desc(1).wait()
        plsc.subcore_barrier()

        rows = NP // NSUB
        @pl.when(cid == 0)
        def _():
            pltpu.sync_copy(acc.at[pl.ds(sid * rows, rows)],
                            out0.at[pl.ds(sid * rows, rows)])

        @pl.when(cid == 1)
        def _():
            pltpu.sync_copy(acc.at[pl.ds(sid * rows, rows)],
                            out1.at[pl.ds(sid * rows, rows)])

    return deg_kernel


def _make_msg_kernel(split: bool):
    per_tile = EP // (NSUB * (NCORE if split else 1))
    nwin = per_tile // K
    zrows = 64
    groups = K // 16

    @functools.partial(
        pl.kernel,
        mesh=_MESH,
        out_type=(
            jax.ShapeDtypeStruct((NP, C), jnp.float32),
            jax.ShapeDtypeStruct((NP, C), jnp.float32),
        ),
        scratch_types=[
            pltpu.VMEM((2, K), jnp.int32),
            pltpu.VMEM((2, K), jnp.int32),
            pltpu.VMEM((2, K), jnp.float32),
            pltpu.VMEM((2, K), jnp.int32),
            pltpu.VMEM((2, K, C), jnp.float32),
            pltpu.VMEM((zrows, C), jnp.float32),
            pltpu.VMEM_SHARED((NP, C), jnp.float32),
            pltpu.SemaphoreType.DMA,
            pltpu.SemaphoreType.DMA,
            pltpu.SemaphoreType.DMA,
            pltpu.SemaphoreType.DMA,
            pltpu.SemaphoreType.DMA,
            pltpu.SemaphoreType.DMA,
        ],
        compiler_params=_SC_PARAMS,
    )
    def msg_kernel(hA, hB, src_hbm, dst_hbm, ew_hbm, out0, out1,
                   srcv, dstv, ewv, dsts, rowsv, zv, acc,
                   lsem0, lsem1, gsem0, gsem1, ssem0, ssem1):
        cid = lax.axis_index("c")
        sid = lax.axis_index("s")
        _zero_spmem_rows(zv, acc, sid * (NP // NSUB), NP // NSUB, zrows)
        plsc.subcore_barrier()

        if split:
            ebase = cid * (EP // NCORE) + sid * per_tile
        else:
            ebase = sid * per_tile

        lsems = (lsem0, lsem1)
        gsems = (gsem0, gsem1)
        ssems = (ssem0, ssem1)

        def l_descs(w, b):
            base = ebase + w * K
            return (
                pltpu.make_async_copy(src_hbm.at[pl.ds(base, K)],
                                      srcv.at[b], lsems[b]),
                pltpu.make_async_copy(dst_hbm.at[pl.ds(base, K)],
                                      dstv.at[b], lsems[b]),
                pltpu.make_async_copy(ew_hbm.at[pl.ds(base, K)],
                                      ewv.at[b], lsems[b]),
            )

        def g_start(b):
            if split:
                pltpu.async_copy(hA.at[srcv.at[b]], rowsv.at[b], gsems[b])
            else:
                @pl.when(cid == 0)
                def _():
                    pltpu.async_copy(hA.at[srcv.at[b]], rowsv.at[b], gsems[b])

                @pl.when(cid == 1)
                def _():
                    pltpu.async_copy(hB.at[srcv.at[b]], rowsv.at[b], gsems[b])

        def g_wait(b):
            pltpu.make_async_copy(hA.at[srcv.at[b]], rowsv.at[b],
                                  gsems[b]).wait()

        def s_desc(b):
            return pltpu.make_async_copy(rowsv.at[b], acc.at[dsts.at[b]],
                                         ssems[b])

        def mult(b):
            def grp(g, _):
                e0 = g * 16
                ew16 = ewv[b, pl.ds(e0, 16)]
                dsts[b, pl.ds(e0, 16)] = dstv[b, pl.ds(e0, 16)]
                for j in range(16):
                    row = rowsv[b, e0 + j, :]
                    rowsv[b, e0 + j, :] = row * ew16[j]
                return ()

            lax.fori_loop(0, groups, grp, ())

        for d in l_descs(0, 0):
            d.start()
        for d in l_descs(0, 0):
            d.wait()
        g_start(0)
        for d in l_descs(1, 1):
            d.start()

        def pair(i, _):
            for b in (0, 1):
                w = 2 * i + b
                nb = 1 - b
                g_wait(b)
                mult(b)
                pltpu.async_copy(rowsv.at[b], acc.at[dsts.at[b]], ssems[b],
                                 add=True)

                @pl.when(w + 1 < nwin)
                def _():
                    for d in l_descs(w + 1, nb):
                        d.wait()

                    @pl.when(w >= 1)
                    def _():
                        s_desc(nb).wait()

                    g_start(nb)

                    @pl.when(w + 2 < nwin)
                    def _():
                        for d in l_descs(w + 2, b):
                            d.start()
            return ()

        lax.fori_loop(0, nwin // 2, pair, ())
        s_desc(0).wait()
        s_desc(1).wait()
        plsc.subcore_barrier()

        rows = NP // NSUB
        @pl.when(cid == 0)
        def _():
            pltpu.sync_copy(acc.at[pl.ds(sid * rows, rows)],
                            out0.at[pl.ds(sid * rows, rows)])

        @pl.when(cid == 1)
        def _():
            pltpu.sync_copy(acc.at[pl.ds(sid * rows, rows)],
                            out1.at[pl.ds(sid * rows, rows)])

    return msg_kernel


_DEG = _make_degree_kernel()
_MSG_FULL = _make_msg_kernel(split=False)
_MSG_SPLIT = _make_msg_kernel(split=True)


def _chunk(hp):
    F = hp.shape[1]
    nc = -(-F // C)
    out = []
    for k in range(nc):
        blk = hp[:, k * C:(k + 1) * C]
        if blk.shape[1] < C:
            blk = jnp.pad(blk, ((0, 0), (0, C - blk.shape[1])))
        out.append(blk)
    return out


def _msg_pass(hp, src, dst, ew):
    F = hp.shape[1]
    chunks = _chunk(hp)
    accs = []
    i = 0
    while i < len(chunks):
        if i + 1 < len(chunks):
            o0, o1 = _MSG_FULL(chunks[i], chunks[i + 1], src, dst, ew)
            accs.extend([o0, o1])
            i += 2
        else:
            o0, o1 = _MSG_SPLIT(chunks[i], chunks[i], src, dst, ew)
            accs.append(o0 + o1)
            i += 1
    return jnp.concatenate(accs, axis=1)[:, :F]


def kernel(x, edge_index, edge_weights, W1, b1, W2, b2, W3, b3, W4, b4, W5, b5):
    pad_idx = (jnp.arange(EP - E, dtype=jnp.int32) * 97) % N
    src = jnp.concatenate([edge_index[0].astype(jnp.int32), pad_idx])
    dst = jnp.concatenate([edge_index[1].astype(jnp.int32), pad_idx])
    ew = jnp.concatenate(
        [edge_weights, jnp.zeros((EP - E,), edge_weights.dtype)])

    p0, p1 = _DEG(dst, ew)
    deg = p0 + p1 + 1.0
    dinv = lax.rsqrt(deg)

    xp = jnp.pad(x, ((0, NP - N), (0, 0)))
    h = xp * W1[0][None, :]
    hp = dinv[:, None] * h
    biases = (b1, b2, b3, b4, b5)
    next_W = (W2, W3, W4, W5, None)
    for bl, Wl in zip(biases, next_W):
        acc = _msg_pass(hp, src, dst, ew)
        out = dinv[:, None] * (acc + hp) + bl[None, :]
        if Wl is None:
            return jax.nn.relu(out[:N])
        hp = dinv[:, None] * (out @ Wl)


# speedup vs baseline: 19.7819x; 1.1876x over previous
import functools

import jax
import jax.numpy as jnp
from jax import lax
from jax.experimental import pallas as pl
from jax.experimental.pallas import tpu as pltpu
from jax.experimental.pallas import tpu_sc as plsc

N = 100000
NP = 102400
E = 1600000
EP = 1638400
C = 16
K = 640
KD = 3200
NSUB = 16
NCORE = 2

_MESH = plsc.VectorSubcoreMesh(core_axis_name="c", subcore_axis_name="s")
_SC_PARAMS = pltpu.CompilerParams(use_tc_tiling_on_sc=False)


def _zero_spmem_rows(zv, acc, rbase, nrows, zrows):
    zero16 = jnp.zeros((C,), jnp.float32)

    def fill(i, _):
        zv[i, :] = zero16
        return ()

    lax.fori_loop(0, zrows, fill, ())

    def copy(j, _):
        pltpu.sync_copy(zv, acc.at[pl.ds(rbase + j * zrows, zrows)])
        return ()

    lax.fori_loop(0, nrows // zrows, copy, ())


def _zero_spmem_flat(zv, acc, base, count, zlen):
    zero16 = jnp.zeros((16,), jnp.float32)

    def fill(i, _):
        zv[pl.ds(i * 16, 16)] = zero16
        return ()

    lax.fori_loop(0, zlen // 16, fill, ())

    def copy(j, _):
        pltpu.sync_copy(zv, acc.at[pl.ds(base + j * zlen, zlen)])
        return ()

    lax.fori_loop(0, count // zlen, copy, ())


def _make_degree_kernel():
    per_tile = EP // (NCORE * NSUB)
    nwin = per_tile // KD
    zlen = 1600

    @functools.partial(
        pl.kernel,
        mesh=_MESH,
        out_type=(
            jax.ShapeDtypeStruct((NP,), jnp.float32),
            jax.ShapeDtypeStruct((NP,), jnp.float32),
        ),
        scratch_types=[
            pltpu.VMEM((2, KD), jnp.int32),
            pltpu.VMEM((2, KD), jnp.float32),
            pltpu.VMEM((2, KD), jnp.int32),
            pltpu.VMEM((2, KD), jnp.float32),
            pltpu.VMEM((zlen,), jnp.float32),
            pltpu.VMEM_SHARED((NP,), jnp.float32),
            pltpu.SemaphoreType.DMA,
            pltpu.SemaphoreType.DMA,
            pltpu.SemaphoreType.DMA,
            pltpu.SemaphoreType.DMA,
        ],
        compiler_params=_SC_PARAMS,
    )
    def deg_kernel(dst_hbm, ew_hbm, out0, out1, dstv, ewv, dsts, ews, zv,
                   acc, lsem0, lsem1, ssem0, ssem1):
        cid = lax.axis_index("c")
        sid = lax.axis_index("s")
        _zero_spmem_flat(zv, acc, sid * (NP // NSUB), NP // NSUB, zlen)
        plsc.subcore_barrier()

        ebase = cid * (EP // NCORE) + sid * per_tile
        lsems = (lsem0, lsem1)
        ssems = (ssem0, ssem1)

        def l_descs(w, b):
            base = ebase + w * KD
            return (
                pltpu.make_async_copy(dst_hbm.at[pl.ds(base, KD)],
                                      dstv.at[b], lsems[b]),
                pltpu.make_async_copy(ew_hbm.at[pl.ds(base, KD)],
                                      ewv.at[b], lsems[b]),
            )

        def s_desc(b):
            return pltpu.make_async_copy(ews.at[b], acc.at[dsts.at[b]],
                                         ssems[b])

        for d in l_descs(0, 0):
            d.start()
        for d in l_descs(1, 1):
            d.start()

        def pair(i, _):
            for b in (0, 1):
                w = 2 * i + b
                for d in l_descs(w, b):
                    d.wait()

                @pl.when(w >= 2)
                def _():
                    s_desc(b).wait()

                def cp(j, _):
                    dsts[b, pl.ds(j * 16, 16)] = dstv[b, pl.ds(j * 16, 16)]
                    ews[b, pl.ds(j * 16, 16)] = ewv[b, pl.ds(j * 16, 16)]
                    return ()

                lax.fori_loop(0, KD // 16, cp, ())
                pltpu.async_copy(ews.at[b], acc.at[dsts.at[b]], ssems[b],
                                 add=True)

                @pl.when(w + 2 < nwin)
                def _():
                    for d in l_descs(w + 2, b):
                        d.start()
            return ()

        lax.fori_loop(0, nwin // 2, pair, ())
        s_desc(0).wait()
        s_desc(1).wait()
        plsc.subcore_barrier()

        rows = NP // NSUB
        @pl.when(cid == 0)
        def _():
            pltpu.sync_copy(acc.at[pl.ds(sid * rows, rows)],
                            out0.at[pl.ds(sid * rows, rows)])

        @pl.when(cid == 1)
        def _():
            pltpu.sync_copy(acc.at[pl.ds(sid * rows, rows)],
                            out1.at[pl.ds(sid * rows, rows)])

    return deg_kernel


def _make_msg_kernel(split: bool):
    per_tile = EP // (NSUB * (NCORE if split else 1))
    nwin = per_tile // K
    zrows = 64
    groups = K // 16

    @functools.partial(
        pl.kernel,
        mesh=_MESH,
        out_type=(
            jax.ShapeDtypeStruct((NP, C), jnp.float32),
            jax.ShapeDtypeStruct((NP, C), jnp.float32),
        ),
        scratch_types=[
            pltpu.VMEM((2, K), jnp.int32),
            pltpu.VMEM((2, K), jnp.int32),
            pltpu.VMEM((2, K), jnp.float32),
            pltpu.VMEM((2, K), jnp.int32),
            pltpu.VMEM((2, K, C), jnp.float32),
            pltpu.VMEM((zrows, C), jnp.float32),
            pltpu.VMEM_SHARED((NP, C), jnp.float32),
            pltpu.SemaphoreType.DMA,
            pltpu.SemaphoreType.DMA,
            pltpu.SemaphoreType.DMA,
            pltpu.SemaphoreType.DMA,
            pltpu.SemaphoreType.DMA,
            pltpu.SemaphoreType.DMA,
        ],
        compiler_params=_SC_PARAMS,
    )
    def msg_kernel(hA, hB, src_hbm, dst_hbm, ew_hbm, out0, out1,
                   srcv, dstv, ewv, dsts, rowsv, zv, acc,
                   lsem0, lsem1, gsem0, gsem1, ssem0, ssem1):
        cid = lax.axis_index("c")
        sid = lax.axis_index("s")
        _zero_spmem_rows(zv, acc, sid * (NP // NSUB), NP // NSUB, zrows)
        plsc.subcore_barrier()

        if split:
            ebase = cid * (EP // NCORE) + sid * per_tile
        else:
            ebase = sid * per_tile

        lsems = (lsem0, lsem1)
        gsems = (gsem0, gsem1)
        ssems = (ssem0, ssem1)

        def l_descs(w, b):
            base = ebase + w * K
            return (
                pltpu.make_async_copy(src_hbm.at[pl.ds(base, K)],
                                      srcv.at[b], lsems[b]),
                pltpu.make_async_copy(dst_hbm.at[pl.ds(base, K)],
                                      dstv.at[b], lsems[b]),
                pltpu.make_async_copy(ew_hbm.at[pl.ds(base, K)],
                                      ewv.at[b], lsems[b]),
            )

        def g_start(b):
            if split:
                pltpu.async_copy(hA.at[srcv.at[b]], rowsv.at[b], gsems[b])
            else:
                @pl.when(cid == 0)
                def _():
                    pltpu.async_copy(hA.at[srcv.at[b]], rowsv.at[b], gsems[b])

                @pl.when(cid == 1)
                def _():
                    pltpu.async_copy(hB.at[srcv.at[b]], rowsv.at[b], gsems[b])

        def g_wait(b):
            pltpu.make_async_copy(hA.at[srcv.at[b]], rowsv.at[b],
                                  gsems[b]).wait()

        def s_desc(b):
            return pltpu.make_async_copy(rowsv.at[b], acc.at[dsts.at[b]],
                                         ssems[b])

        def mult(b):
            def grp(g, _):
                e0 = g * 16
                ew16 = ewv[b, pl.ds(e0, 16)]
                dsts[b, pl.ds(e0, 16)] = dstv[b, pl.ds(e0, 16)]
                for j in range(16):
                    row = rowsv[b, e0 + j, :]
                    rowsv[b, e0 + j, :] = row * ew16[j]
                return ()

            lax.fori_loop(0, groups, grp, ())

        for d in l_descs(0, 0):
            d.start()
        for d in l_descs(0, 0):
            d.wait()
        g_start(0)
        for d in l_descs(1, 1):
            d.start()

        def pair(i, _):
            for b in (0, 1):
                w = 2 * i + b
                nb = 1 - b
                g_wait(b)
                @pl.when(w + 1 < nwin)
                def _():
                    for d in l_descs(w + 1, nb):
                        d.wait()

                    @pl.when(w >= 1)
                    def _():
                        s_desc(nb).wait()

                    g_start(nb)

                mult(b)
                pltpu.async_copy(rowsv.at[b], acc.at[dsts.at[b]], ssems[b],
                                 add=True)

                @pl.when(w + 2 < nwin)
                def _():
                    for d in l_descs(w + 2, b):
                        d.start()
            return ()

        lax.fori_loop(0, nwin // 2, pair, ())
        s_desc(0).wait()
        s_desc(1).wait()
        plsc.subcore_barrier()

        rows = NP // NSUB
        @pl.when(cid == 0)
        def _():
            pltpu.sync_copy(acc.at[pl.ds(sid * rows, rows)],
                            out0.at[pl.ds(sid * rows, rows)])

        @pl.when(cid == 1)
        def _():
            pltpu.sync_copy(acc.at[pl.ds(sid * rows, rows)],
                            out1.at[pl.ds(sid * rows, rows)])

    return msg_kernel


_DEG = _make_degree_kernel()
_MSG_FULL = _make_msg_kernel(split=False)
_MSG_SPLIT = _make_msg_kernel(split=True)


def _chunk(hp):
    F = hp.shape[1]
    nc = -(-F // C)
    out = []
    for k in range(nc):
        blk = hp[:, k * C:(k + 1) * C]
        if blk.shape[1] < C:
            blk = jnp.pad(blk, ((0, 0), (0, C - blk.shape[1])))
        out.append(blk)
    return out


def _msg_pass(hp, src, dst, ew):
    F = hp.shape[1]
    chunks = _chunk(hp)
    accs = []
    i = 0
    while i < len(chunks):
        if i + 1 < len(chunks):
            o0, o1 = _MSG_FULL(chunks[i], chunks[i + 1], src, dst, ew)
            accs.extend([o0, o1])
            i += 2
        else:
            o0, o1 = _MSG_SPLIT(chunks[i], chunks[i], src, dst, ew)
            accs.append(o0 + o1)
            i += 1
    return jnp.concatenate(accs, axis=1)[:, :F]


def kernel(x, edge_index, edge_weights, W1, b1, W2, b2, W3, b3, W4, b4, W5, b5):
    pad_idx = (jnp.arange(EP - E, dtype=jnp.int32) * 97) % N
    src = jnp.concatenate([edge_index[0].astype(jnp.int32), pad_idx])
    dst = jnp.concatenate([edge_index[1].astype(jnp.int32), pad_idx])
    ew = jnp.concatenate(
        [edge_weights, jnp.zeros((EP - E,), edge_weights.dtype)])

    p0, p1 = _DEG(dst, ew)
    deg = p0 + p1 + 1.0
    dinv = lax.rsqrt(deg)

    xp = jnp.pad(x, ((0, NP - N), (0, 0)))
    h = xp * W1[0][None, :]
    hp = dinv[:, None] * h
    biases = (b1, b2, b3, b4, b5)
    next_W = (W2, W3, W4, W5, None)
    for bl, Wl in zip(biases, next_W):
        acc = _msg_pass(hp, src, dst, ew)
        out = dinv[:, None] * (acc + hp) + bl[None, :]
        if Wl is None:
            return jax.nn.relu(out[:N])
        hp = dinv[:, None] * (out @ Wl)


# speedup vs baseline: 21.9113x; 1.1076x over previous
import functools

import jax
import jax.numpy as jnp
from jax import lax
from jax.experimental import pallas as pl
from jax.experimental.pallas import tpu as pltpu
from jax.experimental.pallas import tpu_sc as plsc

N = 100000
NP = 102400
E = 1600000
EP = 1638400
C = 16
K = 640
KD = 3200
NSUB = 16
NCORE = 2

_MESH = plsc.VectorSubcoreMesh(core_axis_name="c", subcore_axis_name="s")
_SC_PARAMS = pltpu.CompilerParams(use_tc_tiling_on_sc=False)


def _zero_spmem_rows(zv, acc, rbase, nrows, zrows):
    zero16 = jnp.zeros((C,), jnp.float32)

    def fill(i, _):
        zv[i, :] = zero16
        return ()

    lax.fori_loop(0, zrows, fill, ())

    def copy(j, _):
        pltpu.sync_copy(zv, acc.at[pl.ds(rbase + j * zrows, zrows)])
        return ()

    lax.fori_loop(0, nrows // zrows, copy, ())


def _zero_spmem_flat(zv, acc, base, count, zlen):
    zero16 = jnp.zeros((16,), jnp.float32)

    def fill(i, _):
        zv[pl.ds(i * 16, 16)] = zero16
        return ()

    lax.fori_loop(0, zlen // 16, fill, ())

    def copy(j, _):
        pltpu.sync_copy(zv, acc.at[pl.ds(base + j * zlen, zlen)])
        return ()

    lax.fori_loop(0, count // zlen, copy, ())


def _make_degree_kernel():
    per_tile = EP // (NCORE * NSUB)
    nwin = per_tile // KD
    zlen = 1600

    @functools.partial(
        pl.kernel,
        mesh=_MESH,
        out_type=(
            jax.ShapeDtypeStruct((NP,), jnp.float32),
            jax.ShapeDtypeStruct((NP,), jnp.float32),
        ),
        scratch_types=[
            pltpu.VMEM((2, KD), jnp.int32),
            pltpu.VMEM((2, KD), jnp.float32),
            pltpu.VMEM((2, KD), jnp.int32),
            pltpu.VMEM((2, KD), jnp.float32),
            pltpu.VMEM((zlen,), jnp.float32),
            pltpu.VMEM_SHARED((NP,), jnp.float32),
            pltpu.SemaphoreType.DMA,
            pltpu.SemaphoreType.DMA,
            pltpu.SemaphoreType.DMA,
            pltpu.SemaphoreType.DMA,
        ],
        compiler_params=_SC_PARAMS,
    )
    def deg_kernel(dst_hbm, ew_hbm, out0, out1, dstv, ewv, dsts, ews, zv,
                   acc, lsem0, lsem1, ssem0, ssem1):
        cid = lax.axis_index("c")
        sid = lax.axis_index("s")
        _zero_spmem_flat(zv, acc, sid * (NP // NSUB), NP // NSUB, zlen)
        plsc.subcore_barrier()

        ebase = cid * (EP // NCORE) + sid * per_tile
        lsems = (lsem0, lsem1)
        ssems = (ssem0, ssem1)

        def l_descs(w, b):
            base = ebase + w * KD
            return (
                pltpu.make_async_copy(dst_hbm.at[pl.ds(base, KD)],
                                      dstv.at[b], lsems[b]),
                pltpu.make_async_copy(ew_hbm.at[pl.ds(base, KD)],
                                      ewv.at[b], lsems[b]),
            )

        def s_desc(b):
            return pltpu.make_async_copy(ews.at[b], acc.at[dsts.at[b]],
                                         ssems[b])

        for d in l_descs(0, 0):
            d.start()
        for d in l_descs(1, 1):
            d.start()

        def pair(i, _):
            for b in (0, 1):
                w = 2 * i + b
                for d in l_descs(w, b):
                    d.wait()

                @pl.when(w >= 2)
                def _():
                    s_desc(b).wait()

                def cp(j, _):
                    dsts[b, pl.ds(j * 16, 16)] = dstv[b, pl.ds(j * 16, 16)]
                    ews[b, pl.ds(j * 16, 16)] = ewv[b, pl.ds(j * 16, 16)]
                    return ()

                lax.fori_loop(0, KD // 16, cp, ())
                pltpu.async_copy(ews.at[b], acc.at[dsts.at[b]], ssems[b],
                                 add=True)

                @pl.when(w + 2 < nwin)
                def _():
                    for d in l_descs(w + 2, b):
                        d.start()
            return ()

        lax.fori_loop(0, nwin // 2, pair, ())
        s_desc(0).wait()
        s_desc(1).wait()
        plsc.subcore_barrier()

        rows = NP // NSUB
        @pl.when(cid == 0)
        def _():
            pltpu.sync_copy(acc.at[pl.ds(sid * rows, rows)],
                            out0.at[pl.ds(sid * rows, rows)])

        @pl.when(cid == 1)
        def _():
            pltpu.sync_copy(acc.at[pl.ds(sid * rows, rows)],
                            out1.at[pl.ds(sid * rows, rows)])

    return deg_kernel


def _make_msg_kernel(split: bool):
    per_tile = EP // (NSUB * (NCORE if split else 1))
    nwin = per_tile // K
    zrows = 64
    groups = K // 16

    @functools.partial(
        pl.kernel,
        mesh=_MESH,
        out_type=(
            jax.ShapeDtypeStruct((NP, C), jnp.float32),
            jax.ShapeDtypeStruct((NP, C), jnp.float32),
        ),
        scratch_types=[
            pltpu.VMEM((2, K), jnp.int32),
            pltpu.VMEM((2, K), jnp.int32),
            pltpu.VMEM((2, K), jnp.float32),
            pltpu.VMEM((2, K), jnp.int32),
            pltpu.VMEM((2, K, C), jnp.float32),
            pltpu.VMEM((zrows, C), jnp.float32),
            pltpu.VMEM_SHARED((NP, C), jnp.float32),
            pltpu.SemaphoreType.DMA,
            pltpu.SemaphoreType.DMA,
            pltpu.SemaphoreType.DMA,
            pltpu.SemaphoreType.DMA,
            pltpu.SemaphoreType.DMA,
            pltpu.SemaphoreType.DMA,
        ],
        compiler_params=_SC_PARAMS,
    )
    def msg_kernel(hA, hB, src_hbm, dst_hbm, ew_hbm, out0, out1,
                   srcv, dstv, ewv, dsts, rowsv, zv, acc,
                   lsem0, lsem1, gsem0, gsem1, ssem0, ssem1):
        cid = lax.axis_index("c")
        sid = lax.axis_index("s")
        _zero_spmem_rows(zv, acc, sid * (NP // NSUB), NP // NSUB, zrows)
        plsc.subcore_barrier()

        if split:
            ebase = cid * (EP // NCORE) + sid * per_tile
        else:
            ebase = sid * per_tile

        lsems = (lsem0, lsem1)
        gsems = (gsem0, gsem1)
        ssems = (ssem0, ssem1)

        def l_descs(w, b):
            base = ebase + w * K
            return (
                pltpu.make_async_copy(src_hbm.at[pl.ds(base, K)],
                                      srcv.at[b], lsems[b]),
                pltpu.make_async_copy(dst_hbm.at[pl.ds(base, K)],
                                      dstv.at[b], lsems[b]),
                pltpu.make_async_copy(ew_hbm.at[pl.ds(base, K)],
                                      ewv.at[b], lsems[b]),
            )

        def g_start(b):
            if split:
                pltpu.async_copy(hA.at[srcv.at[b]], rowsv.at[b], gsems[b])
            else:
                @pl.when(cid == 0)
                def _():
                    pltpu.async_copy(hA.at[srcv.at[b]], rowsv.at[b], gsems[b])

                @pl.when(cid == 1)
                def _():
                    pltpu.async_copy(hB.at[srcv.at[b]], rowsv.at[b], gsems[b])

        def g_wait(b):
            pltpu.make_async_copy(hA.at[srcv.at[b]], rowsv.at[b],
                                  gsems[b]).wait()

        def s_desc(b):
            return pltpu.make_async_copy(rowsv.at[b], acc.at[dsts.at[b]],
                                         ssems[b])

        def mult(b):
            def grp(g, _):
                e0 = g * 16
                ew16 = ewv[b, pl.ds(e0, 16)]
                dsts[b, pl.ds(e0, 16)] = dstv[b, pl.ds(e0, 16)]
                for j in range(16):
                    row = rowsv[b, e0 + j, :]
                    rowsv[b, e0 + j, :] = row * ew16[j]
                return ()

            lax.fori_loop(0, groups, grp, ())

        for d in l_descs(0, 0):
            d.start()
        for d in l_descs(0, 0):
            d.wait()
        g_start(0)
        for d in l_descs(1, 1):
            d.start()

        def pair(i, _):
            for b in (0, 1):
                w = 2 * i + b
                nb = 1 - b
                g_wait(b)
                @pl.when(w + 1 < nwin)
                def _():
                    for d in l_descs(w + 1, nb):
                        d.wait()

                    @pl.when(w >= 1)
                    def _():
                        s_desc(nb).wait()

                    g_start(nb)

                mult(b)
                pltpu.async_copy(rowsv.at[b], acc.at[dsts.at[b]], ssems[b],
                                 add=True)

                @pl.when(w + 2 < nwin)
                def _():
                    for d in l_descs(w + 2, b):
                        d.start()
            return ()

        lax.fori_loop(0, nwin // 2, pair, ())
        s_desc(0).wait()
        s_desc(1).wait()
        plsc.subcore_barrier()

        rows = NP // NSUB
        @pl.when(cid == 0)
        def _():
            pltpu.sync_copy(acc.at[pl.ds(sid * rows, rows)],
                            out0.at[pl.ds(sid * rows, rows)])

        @pl.when(cid == 1)
        def _():
            pltpu.sync_copy(acc.at[pl.ds(sid * rows, rows)],
                            out1.at[pl.ds(sid * rows, rows)])

    return msg_kernel


_DEG = _make_degree_kernel()
_MSG_FULL = _make_msg_kernel(split=False)
_MSG_SPLIT = _make_msg_kernel(split=True)


R = 2048
_TCG = (N + R - 1) // R


def _tc_prep(p0, p1, xp, w1p):
    def body(p0_ref, p1_ref, x_ref, w_ref, dinv_ref, h_ref):
        d = lax.rsqrt(p0_ref[:] + p1_ref[:] + 1.0)
        dinv_ref[:] = d
        h_ref[:, :] = d[:, None] * (x_ref[:, :] * w_ref[:][None, :])

    return pl.pallas_call(
        body,
        grid=(_TCG,),
        in_specs=[
            pl.BlockSpec((R,), lambda i: (i,)),
            pl.BlockSpec((R,), lambda i: (i,)),
            pl.BlockSpec((R, 1), lambda i: (i, 0)),
            pl.BlockSpec((C,), lambda i: (0,)),
        ],
        out_specs=[
            pl.BlockSpec((R,), lambda i: (i,)),
            pl.BlockSpec((R, C), lambda i: (i, 0)),
        ],
        out_shape=[
            jax.ShapeDtypeStruct((NP,), jnp.float32),
            jax.ShapeDtypeStruct((NP, C), jnp.float32),
        ],
    )(p0, p1, xp, w1p)


def _make_tc_layer(ncp, ncn, split_last):
    nacc = (ncp + 1) if split_last else ncp

    def body(*refs):
        acc_refs = refs[:nacc]
        hp_refs = refs[nacc:nacc + ncp]
        dinv_ref, bp_ref, wp_ref = refs[nacc + ncp:nacc + ncp + 3]
        out_refs = refs[nacc + ncp + 3:]
        d = dinv_ref[:]
        hfull = jnp.zeros((R, ncn * C), jnp.float32)
        for k in range(ncp):
            if split_last and k == ncp - 1:
                acck = acc_refs[k][:, :] + acc_refs[k + 1][:, :]
            else:
                acck = acc_refs[k][:, :]
            outk = d[:, None] * (acck + hp_refs[k][:, :]) + bp_ref[k, :][None, :]
            hfull = hfull + jnp.dot(outk, wp_ref[k, :, :],
                                    preferred_element_type=jnp.float32)
        hfull = d[:, None] * hfull
        for j in range(ncn):
            out_refs[j][:, :] = hfull[:, j * C:(j + 1) * C]

    cs = pl.BlockSpec((R, C), lambda i: (i, 0))
    return pl.pallas_call(
        body,
        grid=(_TCG,),
        in_specs=(
            [cs] * (nacc + ncp)
            + [
                pl.BlockSpec((R,), lambda i: (i,)),
                pl.BlockSpec((ncp, C), lambda i: (0, 0)),
                pl.BlockSpec((ncp, C, ncn * C), lambda i: (0, 0, 0)),
            ]
        ),
        out_specs=[cs] * ncn,
        out_shape=[jax.ShapeDtypeStruct((NP, C), jnp.float32)] * ncn,
    )


def _tc_final(accs, hps, dinv, bp):
    ncp = 4

    def body(*refs):
        acc_refs = refs[:ncp]
        hp_refs = refs[ncp:2 * ncp]
        dinv_ref, bp_ref = refs[2 * ncp:2 * ncp + 2]
        out_ref = refs[-1]
        d = dinv_ref[:]
        cols = []
        for k in range(ncp):
            outk = (d[:, None] * (acc_refs[k][:, :] + hp_refs[k][:, :])
                    + bp_ref[k, :][None, :])
            cols.append(outk)
        full = jnp.concatenate(cols, axis=1)
        out_ref[:, :] = jax.nn.relu(full[:, :50])

    cs = pl.BlockSpec((R, C), lambda i: (i, 0))
    return pl.pallas_call(
        body,
        grid=(_TCG,),
        in_specs=(
            [cs] * (2 * ncp)
            + [
                pl.BlockSpec((R,), lambda i: (i,)),
                pl.BlockSpec((ncp, C), lambda i: (0, 0)),
            ]
        ),
        out_specs=pl.BlockSpec((R, 50), lambda i: (i, 0)),
        out_shape=jax.ShapeDtypeStruct((N, 50), jnp.float32),
    )(*accs, *hps, dinv, bp)


_TC2 = _make_tc_layer(ncp=1, ncn=2, split_last=True)
_TC3 = _make_tc_layer(ncp=2, ncn=2, split_last=False)
_TC4 = _make_tc_layer(ncp=2, ncn=3, split_last=False)
_TC5 = _make_tc_layer(ncp=3, ncn=4, split_last=True)


def _pad_w(Wl, ncp, ncn):
    Fp, Fn = Wl.shape
    w = jnp.pad(Wl, ((0, ncp * C - Fp), (0, ncn * C - Fn)))
    return w.reshape(ncp, C, ncn * C)


def _pad_b(bl, ncp):
    return jnp.pad(bl, (0, ncp * C - bl.shape[0])).reshape(ncp, C)


def kernel(x, edge_index, edge_weights, W1, b1, W2, b2, W3, b3, W4, b4, W5, b5):
    pad_idx = (jnp.arange(EP - E, dtype=jnp.int32) * 97) % N
    src = jnp.concatenate([edge_index[0].astype(jnp.int32), pad_idx])
    dst = jnp.concatenate([edge_index[1].astype(jnp.int32), pad_idx])
    ew = jnp.concatenate(
        [edge_weights, jnp.zeros((EP - E,), edge_weights.dtype)])
    xp = jnp.pad(x, ((0, NP - N), (0, 0)))

    p0, p1 = _DEG(dst, ew)
    dinv, h1c0 = _tc_prep(p0, p1, xp, jnp.pad(W1[0], (0, C - W1.shape[1])))

    o0, o1 = _MSG_SPLIT(h1c0, h1c0, src, dst, ew)
    h2c0, h2c1 = _TC2(o0, o1, h1c0, dinv, _pad_b(b1, 1), _pad_w(W2, 1, 2))

    a0, a1 = _MSG_FULL(h2c0, h2c1, src, dst, ew)
    h3c0, h3c1 = _TC3(a0, a1, h2c0, h2c1, dinv,
                      _pad_b(b2, 2), _pad_w(W3, 2, 2))

    b0_, b1_ = _MSG_FULL(h3c0, h3c1, src, dst, ew)
    h4c0, h4c1, h4c2 = _TC4(b0_, b1_, h3c0, h3c1, dinv,
                            _pad_b(b3, 2), _pad_w(W4, 2, 3))

    c0, c1 = _MSG_FULL(h4c0, h4c1, src, dst, ew)
    c2a, c2b = _MSG_SPLIT(h4c2, h4c2, src, dst, ew)
    h5c0, h5c1, h5c2, h5c3 = _TC5(c0, c1, c2a, c2b, h4c0, h4c1, h4c2, dinv,
                                  _pad_b(b4, 3), _pad_w(W5, 3, 4))

    d0, d1 = _MSG_FULL(h5c0, h5c1, src, dst, ew)
    d2, d3 = _MSG_FULL(h5c2, h5c3, src, dst, ew)
    return _tc_final((d0, d1, d2, d3), (h5c0, h5c1, h5c2, h5c3), dinv,
                     _pad_b(b5, 4))


# speedup vs baseline: 22.1211x; 1.0096x over previous
import functools

import jax
import jax.numpy as jnp
from jax import lax
from jax.experimental import pallas as pl
from jax.experimental.pallas import tpu as pltpu
from jax.experimental.pallas import tpu_sc as plsc

N = 100000
NP = 102400
E = 1600000
EP = 1638400
C = 16
K = 640
KD = 3200
NSUB = 16
NCORE = 2

_MESH = plsc.VectorSubcoreMesh(core_axis_name="c", subcore_axis_name="s")
_SC_PARAMS = pltpu.CompilerParams(use_tc_tiling_on_sc=False)


def _zero_spmem_rows(zv, acc, rbase, nrows, zrows, zsem):
    zero16 = jnp.zeros((C,), jnp.float32)

    def fill(i, _):
        zv[i, :] = zero16
        return ()

    lax.fori_loop(0, zrows, fill, ())

    ncopy = nrows // zrows
    batch = 10

    def desc(j):
        return pltpu.make_async_copy(
            zv, acc.at[pl.ds(rbase + j * zrows, zrows)], zsem)

    def copyb(g, _):
        for u in range(batch):
            desc(g * batch + u).start()
        for u in range(batch):
            desc(g * batch + u).wait()
        return ()

    assert ncopy % batch == 0
    lax.fori_loop(0, ncopy // batch, copyb, ())


def _zero_spmem_flat(zv, acc, base, count, zlen):
    zero16 = jnp.zeros((16,), jnp.float32)

    def fill(i, _):
        zv[pl.ds(i * 16, 16)] = zero16
        return ()

    lax.fori_loop(0, zlen // 16, fill, ())

    def copy(j, _):
        pltpu.sync_copy(zv, acc.at[pl.ds(base + j * zlen, zlen)])
        return ()

    lax.fori_loop(0, count // zlen, copy, ())


def _make_degree_kernel():
    per_tile = EP // (NCORE * NSUB)
    nwin = per_tile // KD
    zlen = 1600

    @functools.partial(
        pl.kernel,
        mesh=_MESH,
        out_type=(
            jax.ShapeDtypeStruct((NP,), jnp.float32),
            jax.ShapeDtypeStruct((NP,), jnp.float32),
        ),
        scratch_types=[
            pltpu.VMEM((2, KD), jnp.int32),
            pltpu.VMEM((2, KD), jnp.float32),
            pltpu.VMEM((2, KD), jnp.int32),
            pltpu.VMEM((2, KD), jnp.float32),
            pltpu.VMEM((zlen,), jnp.float32),
            pltpu.VMEM_SHARED((NP,), jnp.float32),
            pltpu.SemaphoreType.DMA,
            pltpu.SemaphoreType.DMA,
            pltpu.SemaphoreType.DMA,
            pltpu.SemaphoreType.DMA,
        ],
        compiler_params=_SC_PARAMS,
    )
    def deg_kernel(dst_hbm, ew_hbm, out0, out1, dstv, ewv, dsts, ews, zv,
                   acc, lsem0, lsem1, ssem0, ssem1):
        cid = lax.axis_index("c")
        sid = lax.axis_index("s")
        _zero_spmem_flat(zv, acc, sid * (NP // NSUB), NP // NSUB, zlen)
        plsc.subcore_barrier()

        ebase = cid * (EP // NCORE) + sid * per_tile
        lsems = (lsem0, lsem1)
        ssems = (ssem0, ssem1)

        def l_descs(w, b):
            base = ebase + w * KD
            return (
                pltpu.make_async_copy(dst_hbm.at[pl.ds(base, KD)],
                                      dstv.at[b], lsems[b]),
                pltpu.make_async_copy(ew_hbm.at[pl.ds(base, KD)],
                                      ewv.at[b], lsems[b]),
            )

        def s_desc(b):
            return pltpu.make_async_copy(ews.at[b], acc.at[dsts.at[b]],
                                         ssems[b])

        for d in l_descs(0, 0):
            d.start()
        for d in l_descs(1, 1):
            d.start()

        def pair(i, _):
            for b in (0, 1):
                w = 2 * i + b
                for d in l_descs(w, b):
                    d.wait()

                @pl.when(w >= 2)
                def _():
                    s_desc(b).wait()

                def cp(j, _):
                    dsts[b, pl.ds(j * 16, 16)] = dstv[b, pl.ds(j * 16, 16)]
                    ews[b, pl.ds(j * 16, 16)] = ewv[b, pl.ds(j * 16, 16)]
                    return ()

                lax.fori_loop(0, KD // 16, cp, ())
                pltpu.async_copy(ews.at[b], acc.at[dsts.at[b]], ssems[b],
                                 add=True)

                @pl.when(w + 2 < nwin)
                def _():
                    for d in l_descs(w + 2, b):
                        d.start()
            return ()

        lax.fori_loop(0, nwin // 2, pair, ())
        s_desc(0).wait()
        s_desc(1).wait()
        plsc.subcore_barrier()

        rows = NP // NSUB
        @pl.when(cid == 0)
        def _():
            pltpu.sync_copy(acc.at[pl.ds(sid * rows, rows)],
                            out0.at[pl.ds(sid * rows, rows)])

        @pl.when(cid == 1)
        def _():
            pltpu.sync_copy(acc.at[pl.ds(sid * rows, rows)],
                            out1.at[pl.ds(sid * rows, rows)])

    return deg_kernel


def _make_msg_kernel(split: bool):
    per_tile = EP // (NSUB * (NCORE if split else 1))
    nwin = per_tile // K
    zrows = 64
    groups = K // 16

    @functools.partial(
        pl.kernel,
        mesh=_MESH,
        out_type=(
            jax.ShapeDtypeStruct((NP, C), jnp.float32),
            jax.ShapeDtypeStruct((NP, C), jnp.float32),
        ),
        scratch_types=[
            pltpu.VMEM((2, K), jnp.int32),
            pltpu.VMEM((2, K), jnp.int32),
            pltpu.VMEM((2, K), jnp.float32),
            pltpu.VMEM((2, K), jnp.int32),
            pltpu.VMEM((2, K, C), jnp.float32),
            pltpu.VMEM((zrows, C), jnp.float32),
            pltpu.VMEM_SHARED((NP, C), jnp.float32),
            pltpu.SemaphoreType.DMA,
            pltpu.SemaphoreType.DMA,
            pltpu.SemaphoreType.DMA,
            pltpu.SemaphoreType.DMA,
            pltpu.SemaphoreType.DMA,
            pltpu.SemaphoreType.DMA,
        ],
        compiler_params=_SC_PARAMS,
    )
    def msg_kernel(hA, hB, src_hbm, dst_hbm, ew_hbm, out0, out1,
                   srcv, dstv, ewv, dsts, rowsv, zv, acc,
                   lsem0, lsem1, gsem0, gsem1, ssem0, ssem1):
        cid = lax.axis_index("c")
        sid = lax.axis_index("s")
        _zero_spmem_rows(zv, acc, sid * (NP // NSUB), NP // NSUB, zrows,
                         lsem0)
        plsc.subcore_barrier()

        if split:
            ebase = cid * (EP // NCORE) + sid * per_tile
        else:
            ebase = sid * per_tile

        lsems = (lsem0, lsem1)
        gsems = (gsem0, gsem1)
        ssems = (ssem0, ssem1)

        def l_descs(w, b):
            base = ebase + w * K
            return (
                pltpu.make_async_copy(src_hbm.at[pl.ds(base, K)],
                                      srcv.at[b], lsems[b]),
                pltpu.make_async_copy(dst_hbm.at[pl.ds(base, K)],
                                      dstv.at[b], lsems[b]),
                pltpu.make_async_copy(ew_hbm.at[pl.ds(base, K)],
                                      ewv.at[b], lsems[b]),
            )

        def g_start(b):
            if split:
                pltpu.async_copy(hA.at[srcv.at[b]], rowsv.at[b], gsems[b])
            else:
                @pl.when(cid == 0)
                def _():
                    pltpu.async_copy(hA.at[srcv.at[b]], rowsv.at[b], gsems[b])

                @pl.when(cid == 1)
                def _():
                    pltpu.async_copy(hB.at[srcv.at[b]], rowsv.at[b], gsems[b])

        def g_wait(b):
            pltpu.make_async_copy(hA.at[srcv.at[b]], rowsv.at[b],
                                  gsems[b]).wait()

        def s_desc(b):
            return pltpu.make_async_copy(rowsv.at[b], acc.at[dsts.at[b]],
                                         ssems[b])

        def mult(b):
            def grp(g, _):
                e0 = g * 16
                ew16 = ewv[b, pl.ds(e0, 16)]
                dsts[b, pl.ds(e0, 16)] = dstv[b, pl.ds(e0, 16)]
                for j in range(16):
                    row = rowsv[b, e0 + j, :]
                    rowsv[b, e0 + j, :] = row * ew16[j]
                return ()

            lax.fori_loop(0, groups, grp, ())

        for d in l_descs(0, 0):
            d.start()
        for d in l_descs(0, 0):
            d.wait()
        g_start(0)
        for d in l_descs(1, 1):
            d.start()

        def pair(i, _):
            for b in (0, 1):
                w = 2 * i + b
                nb = 1 - b
                g_wait(b)
                @pl.when(w + 1 < nwin)
                def _():
                    for d in l_descs(w + 1, nb):
                        d.wait()

                    @pl.when(w >= 1)
                    def _():
                        s_desc(nb).wait()

                    g_start(nb)

                mult(b)
                pltpu.async_copy(rowsv.at[b], acc.at[dsts.at[b]], ssems[b],
                                 add=True)

                @pl.when(w + 2 < nwin)
                def _():
                    for d in l_descs(w + 2, b):
                        d.start()
            return ()

        lax.fori_loop(0, nwin // 2, pair, ())
        s_desc(0).wait()
        s_desc(1).wait()
        plsc.subcore_barrier()

        rows = NP // NSUB
        @pl.when(cid == 0)
        def _():
            pltpu.sync_copy(acc.at[pl.ds(sid * rows, rows)],
                            out0.at[pl.ds(sid * rows, rows)])

        @pl.when(cid == 1)
        def _():
            pltpu.sync_copy(acc.at[pl.ds(sid * rows, rows)],
                            out1.at[pl.ds(sid * rows, rows)])

    return msg_kernel


_DEG = _make_degree_kernel()
_MSG_FULL = _make_msg_kernel(split=False)
_MSG_SPLIT = _make_msg_kernel(split=True)


R = 2048
_TCG = (N + R - 1) // R


def _tc_prep(p0, p1, xp, w1p):
    def body(p0_ref, p1_ref, x_ref, w_ref, dinv_ref, h_ref):
        d = lax.rsqrt(p0_ref[:] + p1_ref[:] + 1.0)
        dinv_ref[:] = d
        h_ref[:, :] = d[:, None] * (x_ref[:, :] * w_ref[:][None, :])

    return pl.pallas_call(
        body,
        grid=(_TCG,),
        in_specs=[
            pl.BlockSpec((R,), lambda i: (i,)),
            pl.BlockSpec((R,), lambda i: (i,)),
            pl.BlockSpec((R, 1), lambda i: (i, 0)),
            pl.BlockSpec((C,), lambda i: (0,)),
        ],
        out_specs=[
            pl.BlockSpec((R,), lambda i: (i,)),
            pl.BlockSpec((R, C), lambda i: (i, 0)),
        ],
        out_shape=[
            jax.ShapeDtypeStruct((NP,), jnp.float32),
            jax.ShapeDtypeStruct((NP, C), jnp.float32),
        ],
    )(p0, p1, xp, w1p)


def _make_tc_layer(ncp, ncn, split_last):
    nacc = (ncp + 1) if split_last else ncp

    def body(*refs):
        acc_refs = refs[:nacc]
        hp_refs = refs[nacc:nacc + ncp]
        dinv_ref, bp_ref, wp_ref = refs[nacc + ncp:nacc + ncp + 3]
        out_refs = refs[nacc + ncp + 3:]
        d = dinv_ref[:]
        hfull = jnp.zeros((R, ncn * C), jnp.float32)
        for k in range(ncp):
            if split_last and k == ncp - 1:
                acck = acc_refs[k][:, :] + acc_refs[k + 1][:, :]
            else:
                acck = acc_refs[k][:, :]
            outk = d[:, None] * (acck + hp_refs[k][:, :]) + bp_ref[k, :][None, :]
            hfull = hfull + jnp.dot(outk, wp_ref[k, :, :],
                                    preferred_element_type=jnp.float32)
        hfull = d[:, None] * hfull
        for j in range(ncn):
            out_refs[j][:, :] = hfull[:, j * C:(j + 1) * C]

    cs = pl.BlockSpec((R, C), lambda i: (i, 0))
    return pl.pallas_call(
        body,
        grid=(_TCG,),
        in_specs=(
            [cs] * (nacc + ncp)
            + [
                pl.BlockSpec((R,), lambda i: (i,)),
                pl.BlockSpec((ncp, C), lambda i: (0, 0)),
                pl.BlockSpec((ncp, C, ncn * C), lambda i: (0, 0, 0)),
            ]
        ),
        out_specs=[cs] * ncn,
        out_shape=[jax.ShapeDtypeStruct((NP, C), jnp.float32)] * ncn,
    )


def _tc_final(accs, hps, dinv, bp):
    ncp = 4

    def body(*refs):
        acc_refs = refs[:ncp]
        hp_refs = refs[ncp:2 * ncp]
        dinv_ref, bp_ref = refs[2 * ncp:2 * ncp + 2]
        out_ref = refs[-1]
        d = dinv_ref[:]
        cols = []
        for k in range(ncp):
            outk = (d[:, None] * (acc_refs[k][:, :] + hp_refs[k][:, :])
                    + bp_ref[k, :][None, :])
            cols.append(outk)
        full = jnp.concatenate(cols, axis=1)
        out_ref[:, :] = jax.nn.relu(full[:, :50])

    cs = pl.BlockSpec((R, C), lambda i: (i, 0))
    return pl.pallas_call(
        body,
        grid=(_TCG,),
        in_specs=(
            [cs] * (2 * ncp)
            + [
                pl.BlockSpec((R,), lambda i: (i,)),
                pl.BlockSpec((ncp, C), lambda i: (0, 0)),
            ]
        ),
        out_specs=pl.BlockSpec((R, 50), lambda i: (i, 0)),
        out_shape=jax.ShapeDtypeStruct((N, 50), jnp.float32),
    )(*accs, *hps, dinv, bp)


_TC2 = _make_tc_layer(ncp=1, ncn=2, split_last=True)
_TC3 = _make_tc_layer(ncp=2, ncn=2, split_last=False)
_TC4 = _make_tc_layer(ncp=2, ncn=3, split_last=False)
_TC5 = _make_tc_layer(ncp=3, ncn=4, split_last=True)


def _pad_w(Wl, ncp, ncn):
    Fp, Fn = Wl.shape
    w = jnp.pad(Wl, ((0, ncp * C - Fp), (0, ncn * C - Fn)))
    return w.reshape(ncp, C, ncn * C)


def _pad_b(bl, ncp):
    return jnp.pad(bl, (0, ncp * C - bl.shape[0])).reshape(ncp, C)


def kernel(x, edge_index, edge_weights, W1, b1, W2, b2, W3, b3, W4, b4, W5, b5):
    pad_idx = (jnp.arange(EP - E, dtype=jnp.int32) * 97) % N
    src = jnp.concatenate([edge_index[0].astype(jnp.int32), pad_idx])
    dst = jnp.concatenate([edge_index[1].astype(jnp.int32), pad_idx])
    ew = jnp.concatenate(
        [edge_weights, jnp.zeros((EP - E,), edge_weights.dtype)])
    xp = jnp.pad(x, ((0, NP - N), (0, 0)))

    p0, p1 = _DEG(dst, ew)
    dinv, h1c0 = _tc_prep(p0, p1, xp, jnp.pad(W1[0], (0, C - W1.shape[1])))

    o0, o1 = _MSG_SPLIT(h1c0, h1c0, src, dst, ew)
    h2c0, h2c1 = _TC2(o0, o1, h1c0, dinv, _pad_b(b1, 1), _pad_w(W2, 1, 2))

    a0, a1 = _MSG_FULL(h2c0, h2c1, src, dst, ew)
    h3c0, h3c1 = _TC3(a0, a1, h2c0, h2c1, dinv,
                      _pad_b(b2, 2), _pad_w(W3, 2, 2))

    b0_, b1_ = _MSG_FULL(h3c0, h3c1, src, dst, ew)
    h4c0, h4c1, h4c2 = _TC4(b0_, b1_, h3c0, h3c1, dinv,
                            _pad_b(b3, 2), _pad_w(W4, 2, 3))

    c0, c1 = _MSG_FULL(h4c0, h4c1, src, dst, ew)
    c2a, c2b = _MSG_SPLIT(h4c2, h4c2, src, dst, ew)
    h5c0, h5c1, h5c2, h5c3 = _TC5(c0, c1, c2a, c2b, h4c0, h4c1, h4c2, dinv,
                                  _pad_b(b4, 3), _pad_w(W5, 3, 4))

    d0, d1 = _MSG_FULL(h5c0, h5c1, src, dst, ew)
    d2, d3 = _MSG_FULL(h5c2, h5c3, src, dst, ew)
    return _tc_final((d0, d1, d2, d3), (h5c0, h5c1, h5c2, h5c3), dinv,
                     _pad_b(b5, 4))
